# baseline (speedup 1.0000x reference)
import functools

import jax
import jax.numpy as jnp
from jax import lax
from jax.experimental import pallas as pl
from jax.experimental.pallas import tpu as pltpu
from jax.experimental.pallas import tpu_sc as plsc

B = 16384
D = 192
NC = 2
NS = 16
L = 16
NW = NC * NS
W = B // NW
FB = 48


@functools.partial(
    pl.kernel,
    mesh=plsc.VectorSubcoreMesh(core_axis_name="c", subcore_axis_name="s"),
    out_type=[
        jax.ShapeDtypeStruct((B,), jnp.float32),
        jax.ShapeDtypeStruct((D, B), jnp.float32),
        jax.ShapeDtypeStruct((D, B), jnp.float32),
    ],
    scratch_types=[
        pltpu.VMEM((FB, W), jnp.float32),
        pltpu.VMEM((FB, W), jnp.float32),
        pltpu.VMEM((W,), jnp.float32),
    ],
)
def _sc_kernel(guT, giT, xui, ouT, oiT, ubuf, vbuf, acc):
    wid = lax.axis_index("s") * NC + lax.axis_index("c")
    base = wid * W
    csl = pl.ds(base, W)
    for j in range(W // L):
        acc[pl.ds(j * L, L)] = jnp.zeros((L,), jnp.float32)
    for fb in range(D // FB):
        fsl = pl.ds(fb * FB, FB)
        pltpu.sync_copy(guT.at[fsl, csl], ubuf)
        pltpu.sync_copy(giT.at[fsl, csl], vbuf)
        pltpu.sync_copy(ubuf, ouT.at[fsl, csl])
        pltpu.sync_copy(vbuf, oiT.at[fsl, csl])

        def body(f, carry):
            for j in range(W // L):
                sl = pl.ds(j * L, L)
                u = ubuf[f, sl]
                v = vbuf[f, sl]
                plsc.addupdate(acc.at[sl], u * v)
            return carry

        lax.fori_loop(0, FB, body, 0)
    pltpu.sync_copy(acc, xui.at[csl])


def kernel(gu, gi):
    xui, ouT, oiT = _sc_kernel(gu.T, gi.T)
    return xui, ouT.T, oiT.T


# speedup vs baseline: 1.8563x; 1.8563x over previous
import functools

import jax
import jax.numpy as jnp
from jax import lax
from jax.experimental import pallas as pl
from jax.experimental.pallas import tpu as pltpu
from jax.experimental.pallas import tpu_sc as plsc

B = 16384
D = 192
NC = 2
NS = 16
L = 16
NW = NC * NS
W = B // NW
FB = 32
NFB = D // FB


@functools.partial(
    pl.kernel,
    mesh=plsc.VectorSubcoreMesh(core_axis_name="c", subcore_axis_name="s"),
    out_type=[
        jax.ShapeDtypeStruct((B,), jnp.float32),
        jax.ShapeDtypeStruct((D, B), jnp.float32),
        jax.ShapeDtypeStruct((D, B), jnp.float32),
    ],
    scratch_types=[
        pltpu.VMEM((FB, W), jnp.float32),
        pltpu.VMEM((FB, W), jnp.float32),
        pltpu.VMEM((FB, W), jnp.float32),
        pltpu.VMEM((FB, W), jnp.float32),
        pltpu.VMEM((W,), jnp.float32),
        pltpu.SemaphoreType.DMA,
        pltpu.SemaphoreType.DMA,
    ],
)
def _sc_kernel(guT, giT, xui, ouT, oiT, ub0, vb0, ub1, vb1, acc, sin, sout):
    wid = lax.axis_index("s") * NC + lax.axis_index("c")
    base = wid * W
    csl = pl.ds(base, W)
    ubufs = (ub0, ub1)
    vbufs = (vb0, vb1)

    def in_copies(fb):
        fsl = pl.ds(fb * FB, FB)
        return (
            pltpu.make_async_copy(guT.at[fsl, csl], ubufs[fb % 2], sin),
            pltpu.make_async_copy(giT.at[fsl, csl], vbufs[fb % 2], sin),
        )

    def out_copies(fb):
        fsl = pl.ds(fb * FB, FB)
        return (
            pltpu.make_async_copy(ubufs[fb % 2], ouT.at[fsl, csl], sout),
            pltpu.make_async_copy(vbufs[fb % 2], oiT.at[fsl, csl], sout),
        )

    for fb in (0, 1):
        cu, cv = in_copies(fb)
        cu.start()
        cv.start()

    for fb in range(NFB):
        cu, cv = in_copies(fb)
        cu.wait()
        cv.wait()
        ou, oi = out_copies(fb)
        ou.start()
        oi.start()

        ub = ubufs[fb % 2]
        vb = vbufs[fb % 2]

        def body(j, carry, ub=ub, vb=vb, first=(fb == 0)):
            sl = pl.ds(j * L, L)
            a = jnp.zeros((L,), jnp.float32) if first else acc[sl]
            for f in range(FB):
                a = a + ub[f, sl] * vb[f, sl]
            acc[sl] = a
            return carry

        lax.fori_loop(0, W // L, body, 0)

        if fb + 2 < NFB:
            ou.wait()
            oi.wait()
            nu, nv = in_copies(fb + 2)
            nu.start()
            nv.start()

    for fb in (NFB - 2, NFB - 1):
        ou, oi = out_copies(fb)
        ou.wait()
        oi.wait()

    pltpu.sync_copy(acc, xui.at[csl])


def kernel(gu, gi):
    xui, ouT, oiT = _sc_kernel(gu.T, gi.T)
    return xui, ouT.T, oiT.T


# speedup vs baseline: 2.0230x; 1.0898x over previous
import functools

import jax
import jax.numpy as jnp
from jax import lax
from jax.experimental import pallas as pl
from jax.experimental.pallas import tpu as pltpu
from jax.experimental.pallas import tpu_sc as plsc

B = 16384
D = 192
NC = 2
NS = 16
NW = NC * NS
RG = 8
CC = 4096
NCH = (D // RG) * (B // CC)
CPW = NCH // NW


@functools.partial(
    pl.kernel,
    mesh=plsc.VectorSubcoreMesh(core_axis_name="c", subcore_axis_name="s"),
    out_type=jax.ShapeDtypeStruct((D, B), jnp.float32),
    scratch_types=[
        pltpu.VMEM((RG, CC), jnp.float32),
        pltpu.VMEM((RG, CC), jnp.float32),
        pltpu.VMEM((RG, CC), jnp.float32),
        pltpu.SemaphoreType.DMA,
        pltpu.SemaphoreType.DMA,
    ],
)
def _sc_copy(giT, oiT, b0, b1, b2, sin, sout):
    wid = lax.axis_index("s") * NC + lax.axis_index("c")
    bufs = (b0, b1, b2)

    def chunk_slices(k):
        c = wid * CPW + k
        rg = c // (B // CC)
        cc = c % (B // CC)
        return pl.ds(rg * RG, RG), pl.ds(cc * CC, CC)

    for k in range(CPW):
        rsl, csl = chunk_slices(k)
        pltpu.make_async_copy(giT.at[rsl, csl], bufs[k], sin).start()
    for k in range(CPW):
        rsl, csl = chunk_slices(k)
        pltpu.make_async_copy(giT.at[rsl, csl], bufs[k], sin).wait()
        pltpu.make_async_copy(bufs[k], oiT.at[rsl, csl], sout).start()
    for k in range(CPW):
        rsl, csl = chunk_slices(k)
        pltpu.make_async_copy(bufs[k], oiT.at[rsl, csl], sout).wait()


def _tc_body(guT_ref, giT_ref, xui_ref, uT_ref):
    u = guT_ref[...]
    v = giT_ref[...]
    uT_ref[...] = u
    xui_ref[...] = jnp.sum(u * v, axis=0)


def kernel(gu, gi):
    guT = gu.T
    giT = gi.T
    gamma_iT = _sc_copy(giT)
    BS = 2048
    xui, gamma_uT = pl.pallas_call(
        _tc_body,
        grid=(B // BS,),
        in_specs=[
            pl.BlockSpec((D, BS), lambda b: (0, b)),
            pl.BlockSpec((D, BS), lambda b: (0, b)),
        ],
        out_specs=[
            pl.BlockSpec((BS,), lambda b: (b,)),
            pl.BlockSpec((D, BS), lambda b: (0, b)),
        ],
        out_shape=[
            jax.ShapeDtypeStruct((B,), gu.dtype),
            jax.ShapeDtypeStruct((D, B), gu.dtype),
        ],
    )(guT, giT)
    return (xui, gamma_uT.T, gamma_iT.T)
